# Initial kernel scaffold; baseline (speedup 1.0000x reference)
import jax
import jax.numpy as jnp
from jax.experimental import pallas as pl


def kernel(input, target):
    raise NotImplementedError("write your pallas kernel here")


# speedup vs baseline: 184.0631x; 184.0631x over previous
import functools

import numpy as np
import jax
import jax.numpy as jnp
from jax import lax
from jax.experimental import pallas as pl
from jax.experimental.pallas import tpu as pltpu
from jax.experimental.pallas import tpu_sc as plsc

V = 100
N = 12288
ALPHA = 0.1
SMOOTH_TAIL = 0.01

_cls_num = np.array([100000 - 1000 * i for i in range(100)], dtype=np.float64)
_CLS_SMOOTH = (SMOOTH_TAIL + (ALPHA - SMOOTH_TAIL)
               * (_cls_num - _cls_num.min()) / (_cls_num.max() - _cls_num.min()))
_A = float(_CLS_SMOOTH.sum())
_B = float((_CLS_SMOOTH * np.log(_CLS_SMOOTH)).sum())
_LOG_V1 = float(np.log(V - 1))
_LN09 = float(np.log(1.0 - ALPHA))


_RB = 1024


def _row_stats_body(x_ref, t_ref, s_ref, s1_ref, s2_ref, lpt_ref, st_ref,
                    pred_ref):
    x = x_ref[...]
    t = t_ref[...]
    s = s_ref[...]
    m = jnp.max(x, axis=1, keepdims=True)
    z = m + jnp.log(jnp.sum(jnp.exp(x - m), axis=1, keepdims=True))
    t1 = jnp.sum(x, axis=1, keepdims=True)
    t2 = jnp.sum(x * s, axis=1, keepdims=True)
    lanes = lax.broadcasted_iota(jnp.int32, x.shape, 1)
    onehot = lanes == t
    xt = jnp.sum(jnp.where(onehot, x, 0.0), axis=1, keepdims=True)
    st = jnp.sum(jnp.where(onehot, s, 0.0), axis=1, keepdims=True)
    pred = jnp.min(jnp.where(x == m, lanes, V), axis=1, keepdims=True)
    s1_ref[...] = t1 - V * z
    s2_ref[...] = t2 - _A * z
    lpt_ref[...] = xt - z
    st_ref[...] = st
    pred_ref[...] = pred


def _row_stats(x, t2, s2d):
    f = jax.ShapeDtypeStruct((N, 1), jnp.float32)
    return pl.pallas_call(
        _row_stats_body,
        grid=(N // _RB,),
        in_specs=[
            pl.BlockSpec((_RB, V), lambda i: (i, 0)),
            pl.BlockSpec((_RB, 1), lambda i: (i, 0)),
            pl.BlockSpec((1, V), lambda i: (0, 0)),
        ],
        out_specs=[pl.BlockSpec((_RB, 1), lambda i: (i, 0))] * 5,
        out_shape=[f, f, f, f, jax.ShapeDtypeStruct((N, 1), jnp.int32)],
    )(x, t2, s2d)


_LN = 16
_NT = 16
_NW = 32
_SL = N // _NW
_SEGC = N // _NT
_RPAD = N + 32


def _sc_body(t_hbm, pred_hbm, out_hbm, t_v, pred_v, tk_v, pk_v, segid_v,
             mr_v, row_v, ed_v, es_v, estab_v, out_v, es_sh):
    cid = lax.axis_index("c")
    sid = lax.axis_index("s")
    k = sid
    iota = lax.iota(jnp.int32, _LN)

    pltpu.sync_copy(t_hbm, t_v.at[pl.ds(0, N)])
    pltpu.sync_copy(pred_hbm, pred_v.at[pl.ds(0, N)])

    def compact_body(c, carry):
        nrun, mrun = carry
        t16 = t_v[pl.ds(c * _LN, _LN)]
        p16 = pred_v[pl.ds(c * _LN, _LN)]
        mask = t16 != 0
        mask3 = t16 == 3
        mi = mask.astype(jnp.int32)
        m3i = mask3.astype(jnp.int32)
        cnt = jnp.max(plsc.all_reduce_population_count(mask))
        cnt3 = jnp.max(plsc.all_reduce_population_count(mask3))
        plsc.store_compressed(tk_v.at[pl.ds(nrun, _LN)], t16, mask=mask)
        plsc.store_compressed(pk_v.at[pl.ds(nrun, _LN)], p16, mask=mask)
        ranks = nrun + plsc.cumsum(mi) - 1
        plsc.store_compressed(mr_v.at[pl.ds(mrun, _LN)], ranks, mask=mask3)
        segid_v[pl.ds(c * _LN, _LN)] = mrun + plsc.cumsum(m3i) - m3i
        return nrun + cnt, mrun + cnt3

    _, nseg_total = lax.fori_loop(0, N // _LN, compact_body,
                                  (jnp.int32(0), jnp.int32(0)))

    def _sload(ref, idx):
        return plsc.load_gather(ref, [jnp.full((_LN,), idx, jnp.int32)])[0]

    def seg_body(si, _):
        sgi = k + si * _NT
        prev_idx = jnp.maximum(sgi - 1, 0)
        bounds = plsc.load_gather(
            mr_v, [jnp.where(iota < 1, sgi, prev_idx)])
        end = bounds[0]
        start = jnp.where(sgi == 0, 0, bounds[1] + 1)
        seg_l = end - start
        nch = (seg_l + _LN - 1) // _LN

        def init_body(c, _c):
            row_v[pl.ds(c * _LN, _LN)] = c * _LN + iota
            return 0

        lax.fori_loop(0, nch, init_body, 0)

        def row_body(i, _c):
            src = ((i - 1) & 1) * _RPAD
            dst = (i & 1) * _RPAD
            ai = _sload(tk_v, start + i - 1)
            row_v[pl.ds(dst, _LN)] = jnp.full((_LN,), i, jnp.int32)

            def ch_body(c, carry):
                j0 = c * _LN
                p = row_v[pl.ds(src + j0 + 1, _LN)]
                pm1 = row_v[pl.ds(src + j0, _LN)]
                bv = pk_v[pl.ds(start + j0, _LN)]
                cost = (bv != ai).astype(jnp.int32)
                mmin = jnp.minimum(p + 1, pm1 + cost)
                jvec = j0 + 1 + iota
                u = mmin - jvec
                pf = -plsc.cummax(-u)
                row_v[pl.ds(dst + j0 + 1, _LN)] = \
                    jvec + jnp.minimum(carry, pf)
                return jnp.minimum(carry, jnp.min(u))

            lax.fori_loop(0, nch, ch_body, i)
            return 0

        lax.fori_loop(1, seg_l + 1, row_body, 0)
        ed = _sload(row_v, (seg_l & 1) * _RPAD + seg_l)
        ed = jnp.where(seg_l == 0, 0, ed)
        plsc.store_scatter(ed_v, [jnp.full((_LN,), si, jnp.int32)],
                           jnp.full((_LN,), ed, jnp.int32))
        return 0

    nseg_mine = (jnp.maximum(nseg_total - k, 0) + _NT - 1) // _NT
    lax.fori_loop(0, nseg_mine, seg_body, 0)

    def es_body(c, _c):
        ed16 = ed_v[pl.ds(c * _LN, _LN)]
        edf = jnp.maximum(ed16.astype(jnp.float32), 1.0)
        es16 = jnp.where(ed16 != 0, 1.0 - jnp.exp(_LN09 / edf),
                         jnp.float32(1e-12))
        es_v[pl.ds(c * _LN, _LN)] = es16
        return 0

    lax.fori_loop(0, _SEGC // _LN, es_body, 0)
    pltpu.sync_copy(es_v, es_sh.at[k])
    plsc.subcore_barrier()

    pltpu.sync_copy(es_sh, estab_v)
    base = (cid * _NT + sid) * _SL

    def out_body(c, _c):
        off = base + c * _LN
        t16 = t_v[pl.ds(off, _LN)]
        sg16 = segid_v[pl.ds(off, _LN)]
        okseg = sg16 < nseg_total
        sgc = jnp.where(okseg, sg16, 0)
        es16 = plsc.load_gather(estab_v, [sgc & (_NT - 1), sgc >> 4])
        act = okseg & (t16 != 0) & (t16 != 3)
        out_v[pl.ds(c * _LN, _LN)] = jnp.where(act, es16, 0.0)
        return 0

    lax.fori_loop(0, _SL // _LN, out_body, 0)
    pltpu.sync_copy(out_v, out_hbm.at[pl.ds(base, _SL)])


@functools.cache
def _sc_es_call():
  return functools.partial(
    pl.kernel,
    out_type=jax.ShapeDtypeStruct((N,), jnp.float32),
    compiler_params=pltpu.CompilerParams(needs_layout_passes=False,
                                         use_tc_tiling_on_sc=False),
    mesh=plsc.VectorSubcoreMesh(core_axis_name="c", subcore_axis_name="s",
                                num_cores=2, num_subcores=16),
    scratch_types=[
        pltpu.VMEM((N + _LN,), jnp.int32),
        pltpu.VMEM((N + _LN,), jnp.int32),
        pltpu.VMEM((N + _LN,), jnp.int32),
        pltpu.VMEM((N + _LN,), jnp.int32),
        pltpu.VMEM((N,), jnp.int32),
        pltpu.VMEM((N,), jnp.int32),
        pltpu.VMEM((2 * _RPAD,), jnp.int32),
        pltpu.VMEM((_SEGC,), jnp.int32),
        pltpu.VMEM((_SEGC,), jnp.float32),
        pltpu.VMEM((_NT, _SEGC), jnp.float32),
        pltpu.VMEM((_SL,), jnp.float32),
        pltpu.VMEM_SHARED((_NT, _SEGC), jnp.float32),
    ],
  )(_sc_body)


def _combine_body(t_ref, s1_ref, s2_ref, lpt_ref, st_ref, es_ref, out_ref):
    t = t_ref[...]
    s1 = s1_ref[...]
    s2 = s2_ref[...]
    lpt = lpt_ref[...]
    st = st_ref[...]
    es = es_ref[...]
    valid = t != 0
    active = es > 0.0
    head = (t >= 4) & (t < 24)
    ess = jnp.where(active, es, 1.0)
    vt = ess * st / (V - 1)
    ch = 1.0 - ess * _A / (V - 1)
    sum_vlogv = (ess / (V - 1)) * (_A * (jnp.log(ess) - _LOG_V1) + _B)
    l_head = (sum_vlogv - vt * jnp.log(vt)) \
        - ((ess / (V - 1)) * s2 - vt * lpt) \
        + ch * jnp.log(ch) - ch * lpt
    kk = ess * jnp.float32(1e-12) / (V - 1)
    cn = 1.0 - V * kk
    l_tail = (V - 1) * kk * jnp.log(kk) - kk * (s1 - lpt) \
        + cn * jnp.log(cn) - cn * lpt
    l_in = -lpt
    l_row = jnp.where(active, jnp.where(head, l_head, l_tail), l_in)
    l_row = jnp.where(valid, l_row, 0.0)
    n = jnp.sum(valid.astype(jnp.float32), axis=(0, 1), keepdims=True)
    out_ref[...] = jnp.sum(l_row, axis=(0, 1), keepdims=True) / n


def _combine(t2, s1, s2, lpt, st, es2):
    return pl.pallas_call(
        _combine_body,
        out_shape=jax.ShapeDtypeStruct((1, 1), jnp.float32),
    )(t2, s1, s2, lpt, st, es2)


def kernel(input, target):
    x = jnp.reshape(input, (N, V))
    t1 = jnp.reshape(target, (N,)).astype(jnp.int32)
    t2 = jnp.reshape(t1, (N, 1))
    s2d = jnp.asarray(_CLS_SMOOTH.reshape(1, V), dtype=jnp.float32)
    s1, s2, lpt, st, pred = _row_stats(x, t2, s2d)
    es = _sc_es_call()(t1, jnp.reshape(pred, (N,)))
    sq = lambda a: jnp.reshape(a, (96, 128))
    out = _combine(sq(t2), sq(s1), sq(s2), sq(lpt), sq(st), sq(es))
    return out[0, 0]


# speedup vs baseline: 187.4159x; 1.0182x over previous
import functools

import numpy as np
import jax
import jax.numpy as jnp
from jax import lax
from jax.experimental import pallas as pl
from jax.experimental.pallas import tpu as pltpu
from jax.experimental.pallas import tpu_sc as plsc

V = 100
N = 12288
ALPHA = 0.1
SMOOTH_TAIL = 0.01

_cls_num = np.array([100000 - 1000 * i for i in range(100)], dtype=np.float64)
_CLS_SMOOTH = (SMOOTH_TAIL + (ALPHA - SMOOTH_TAIL)
               * (_cls_num - _cls_num.min()) / (_cls_num.max() - _cls_num.min()))
_A = float(_CLS_SMOOTH.sum())
_B = float((_CLS_SMOOTH * np.log(_CLS_SMOOTH)).sum())
_LOG_V1 = float(np.log(V - 1))
_LN09 = float(np.log(1.0 - ALPHA))


_RB = 1024


def _row_stats_body(x_ref, t_ref, s_ref, s1_ref, s2_ref, lpt_ref, st_ref,
                    pred_ref):
    x = x_ref[...]
    t = t_ref[...]
    s = s_ref[...]
    m = jnp.max(x, axis=1, keepdims=True)
    z = m + jnp.log(jnp.sum(jnp.exp(x - m), axis=1, keepdims=True))
    t1 = jnp.sum(x, axis=1, keepdims=True)
    t2 = jnp.sum(x * s, axis=1, keepdims=True)
    lanes = lax.broadcasted_iota(jnp.int32, x.shape, 1)
    onehot = lanes == t
    xt = jnp.sum(jnp.where(onehot, x, 0.0), axis=1, keepdims=True)
    st = jnp.sum(jnp.where(onehot, s, 0.0), axis=1, keepdims=True)
    pred = jnp.min(jnp.where(x == m, lanes, V), axis=1, keepdims=True)
    s1_ref[...] = t1 - V * z
    s2_ref[...] = t2 - _A * z
    lpt_ref[...] = xt - z
    st_ref[...] = st
    pred_ref[...] = pred


def _row_stats(x, t2, s2d):
    f = jax.ShapeDtypeStruct((N, 1), jnp.float32)
    return pl.pallas_call(
        _row_stats_body,
        grid=(N // _RB,),
        in_specs=[
            pl.BlockSpec((_RB, V), lambda i: (i, 0)),
            pl.BlockSpec((_RB, 1), lambda i: (i, 0)),
            pl.BlockSpec((1, V), lambda i: (0, 0)),
        ],
        out_specs=[pl.BlockSpec((_RB, 1), lambda i: (i, 0))] * 5,
        out_shape=[f, f, f, f, jax.ShapeDtypeStruct((N, 1), jnp.int32)],
    )(x, t2, s2d)


_LN = 16
_NT = 16
_NW = 32
_SL = N // _NW
_SEGC = N // _NT
_RPAD = N + 32


def _sc_body(t_hbm, pred_hbm, out_hbm, t_v, pred_v, tk_v, pk_v, segid_v,
             mr_v, row_v, ed_v, es_v, estab_v, out_v, es_sh):
    cid = lax.axis_index("c")
    sid = lax.axis_index("s")
    k = sid
    iota = lax.iota(jnp.int32, _LN)

    pltpu.sync_copy(t_hbm, t_v.at[pl.ds(0, N)])
    pltpu.sync_copy(pred_hbm, pred_v.at[pl.ds(0, N)])

    def compact_body(c, carry):
        nrun, mrun = carry
        t16 = t_v[pl.ds(c * _LN, _LN)]
        p16 = pred_v[pl.ds(c * _LN, _LN)]
        mask = t16 != 0
        mask3 = t16 == 3
        mi = mask.astype(jnp.int32)
        m3i = mask3.astype(jnp.int32)
        cnt = jnp.max(plsc.all_reduce_population_count(mask))
        cnt3 = jnp.max(plsc.all_reduce_population_count(mask3))
        plsc.store_compressed(tk_v.at[pl.ds(nrun, _LN)], t16, mask=mask)
        plsc.store_compressed(pk_v.at[pl.ds(nrun, _LN)], p16, mask=mask)
        ranks = nrun + plsc.cumsum(mi) - 1
        plsc.store_compressed(mr_v.at[pl.ds(mrun, _LN)], ranks, mask=mask3)
        segid_v[pl.ds(c * _LN, _LN)] = mrun + plsc.cumsum(m3i) - m3i
        return nrun + cnt, mrun + cnt3

    _, nseg_total = lax.fori_loop(0, N // _LN, compact_body,
                                  (jnp.int32(0), jnp.int32(0)))

    def _sload(ref, idx):
        return plsc.load_gather(ref, [jnp.full((_LN,), idx, jnp.int32)])[0]

    def seg_body(si, _):
        sgi = k + si * _NT
        prev_idx = jnp.maximum(sgi - 1, 0)
        bounds = plsc.load_gather(
            mr_v, [jnp.where(iota < 1, sgi, prev_idx)])
        end = bounds[0]
        start = jnp.where(sgi == 0, 0, bounds[1] + 1)
        seg_l = end - start
        nch = (seg_l + _LN - 1) // _LN

        def init_body(c, _c):
            row_v[pl.ds(c * _LN, _LN)] = c * _LN + iota
            return 0

        lax.fori_loop(0, nch, init_body, 0)

        def row_body(i, _c):
            src = ((i - 1) & 1) * _RPAD
            dst = (i & 1) * _RPAD
            ai = _sload(tk_v, start + i - 1)
            row_v[pl.ds(dst, _LN)] = jnp.full((_LN,), i, jnp.int32)

            def ch_body(c, carry):
                j0 = c * _LN
                p = row_v[pl.ds(src + j0 + 1, _LN)]
                pm1 = row_v[pl.ds(src + j0, _LN)]
                bv = pk_v[pl.ds(start + j0, _LN)]
                cost = (bv != ai).astype(jnp.int32)
                mmin = jnp.minimum(p + 1, pm1 + cost)
                jvec = j0 + 1 + iota
                u = mmin - jvec
                pf = -plsc.cummax(-u)
                pfc = jnp.minimum(carry, pf)
                row_v[pl.ds(dst + j0 + 1, _LN)] = jvec + pfc
                return pfc[_LN - 1]

            lax.fori_loop(0, nch, ch_body, i)
            return 0

        lax.fori_loop(1, seg_l + 1, row_body, 0)
        ed = _sload(row_v, (seg_l & 1) * _RPAD + seg_l)
        ed = jnp.where(seg_l == 0, 0, ed)
        plsc.store_scatter(ed_v, [jnp.full((_LN,), si, jnp.int32)],
                           jnp.full((_LN,), ed, jnp.int32))
        return 0

    nseg_mine = (jnp.maximum(nseg_total - k, 0) + _NT - 1) // _NT
    lax.fori_loop(0, nseg_mine, seg_body, 0)

    def es_body(c, _c):
        ed16 = ed_v[pl.ds(c * _LN, _LN)]
        edf = jnp.maximum(ed16.astype(jnp.float32), 1.0)
        es16 = jnp.where(ed16 != 0, 1.0 - jnp.exp(_LN09 / edf),
                         jnp.float32(1e-12))
        es_v[pl.ds(c * _LN, _LN)] = es16
        return 0

    lax.fori_loop(0, _SEGC // _LN, es_body, 0)
    pltpu.sync_copy(es_v, es_sh.at[k])
    plsc.subcore_barrier()

    pltpu.sync_copy(es_sh, estab_v)
    base = (cid * _NT + sid) * _SL

    def out_body(c, _c):
        off = base + c * _LN
        t16 = t_v[pl.ds(off, _LN)]
        sg16 = segid_v[pl.ds(off, _LN)]
        okseg = sg16 < nseg_total
        sgc = jnp.where(okseg, sg16, 0)
        es16 = plsc.load_gather(estab_v, [sgc & (_NT - 1), sgc >> 4])
        act = okseg & (t16 != 0) & (t16 != 3)
        out_v[pl.ds(c * _LN, _LN)] = jnp.where(act, es16, 0.0)
        return 0

    lax.fori_loop(0, _SL // _LN, out_body, 0)
    pltpu.sync_copy(out_v, out_hbm.at[pl.ds(base, _SL)])


@functools.cache
def _sc_es_call():
  return functools.partial(
    pl.kernel,
    out_type=jax.ShapeDtypeStruct((N,), jnp.float32),
    compiler_params=pltpu.CompilerParams(needs_layout_passes=False,
                                         use_tc_tiling_on_sc=False),
    mesh=plsc.VectorSubcoreMesh(core_axis_name="c", subcore_axis_name="s",
                                num_cores=2, num_subcores=16),
    scratch_types=[
        pltpu.VMEM((N + _LN,), jnp.int32),
        pltpu.VMEM((N + _LN,), jnp.int32),
        pltpu.VMEM((N + _LN,), jnp.int32),
        pltpu.VMEM((N + _LN,), jnp.int32),
        pltpu.VMEM((N,), jnp.int32),
        pltpu.VMEM((N,), jnp.int32),
        pltpu.VMEM((2 * _RPAD,), jnp.int32),
        pltpu.VMEM((_SEGC,), jnp.int32),
        pltpu.VMEM((_SEGC,), jnp.float32),
        pltpu.VMEM((_NT, _SEGC), jnp.float32),
        pltpu.VMEM((_SL,), jnp.float32),
        pltpu.VMEM_SHARED((_NT, _SEGC), jnp.float32),
    ],
  )(_sc_body)


def _combine_body(t_ref, s1_ref, s2_ref, lpt_ref, st_ref, es_ref, out_ref):
    t = t_ref[...]
    s1 = s1_ref[...]
    s2 = s2_ref[...]
    lpt = lpt_ref[...]
    st = st_ref[...]
    es = es_ref[...]
    valid = t != 0
    active = es > 0.0
    head = (t >= 4) & (t < 24)
    ess = jnp.where(active, es, 1.0)
    vt = ess * st / (V - 1)
    ch = 1.0 - ess * _A / (V - 1)
    sum_vlogv = (ess / (V - 1)) * (_A * (jnp.log(ess) - _LOG_V1) + _B)
    l_head = (sum_vlogv - vt * jnp.log(vt)) \
        - ((ess / (V - 1)) * s2 - vt * lpt) \
        + ch * jnp.log(ch) - ch * lpt
    kk = ess * jnp.float32(1e-12) / (V - 1)
    cn = 1.0 - V * kk
    l_tail = (V - 1) * kk * jnp.log(kk) - kk * (s1 - lpt) \
        + cn * jnp.log(cn) - cn * lpt
    l_in = -lpt
    l_row = jnp.where(active, jnp.where(head, l_head, l_tail), l_in)
    l_row = jnp.where(valid, l_row, 0.0)
    n = jnp.sum(valid.astype(jnp.float32), axis=(0, 1), keepdims=True)
    out_ref[...] = jnp.sum(l_row, axis=(0, 1), keepdims=True) / n


def _combine(t2, s1, s2, lpt, st, es2):
    return pl.pallas_call(
        _combine_body,
        out_shape=jax.ShapeDtypeStruct((1, 1), jnp.float32),
    )(t2, s1, s2, lpt, st, es2)


def kernel(input, target):
    x = jnp.reshape(input, (N, V))
    t1 = jnp.reshape(target, (N,)).astype(jnp.int32)
    t2 = jnp.reshape(t1, (N, 1))
    s2d = jnp.asarray(_CLS_SMOOTH.reshape(1, V), dtype=jnp.float32)
    s1, s2, lpt, st, pred = _row_stats(x, t2, s2d)
    es = _sc_es_call()(t1, jnp.reshape(pred, (N,)))
    sq = lambda a: jnp.reshape(a, (96, 128))
    out = _combine(sq(t2), sq(s1), sq(s2), sq(lpt), sq(st), sq(es))
    return out[0, 0]


# speedup vs baseline: 343.8946x; 1.8349x over previous
import functools

import numpy as np
import jax
import jax.numpy as jnp
from jax import lax
from jax.experimental import pallas as pl
from jax.experimental.pallas import tpu as pltpu
from jax.experimental.pallas import tpu_sc as plsc

V = 100
N = 12288
ALPHA = 0.1
SMOOTH_TAIL = 0.01

_cls_num = np.array([100000 - 1000 * i for i in range(100)], dtype=np.float64)
_CLS_SMOOTH = (SMOOTH_TAIL + (ALPHA - SMOOTH_TAIL)
               * (_cls_num - _cls_num.min()) / (_cls_num.max() - _cls_num.min()))
_A = float(_CLS_SMOOTH.sum())
_B = float((_CLS_SMOOTH * np.log(_CLS_SMOOTH)).sum())
_LOG_V1 = float(np.log(V - 1))
_LN09 = float(np.log(1.0 - ALPHA))


_RB = 1024


def _row_stats_body(x_ref, t_ref, s_ref, s1_ref, s2_ref, lpt_ref, st_ref,
                    pred_ref):
    x = x_ref[...]
    t = t_ref[...]
    s = s_ref[...]
    m = jnp.max(x, axis=1, keepdims=True)
    z = m + jnp.log(jnp.sum(jnp.exp(x - m), axis=1, keepdims=True))
    t1 = jnp.sum(x, axis=1, keepdims=True)
    t2 = jnp.sum(x * s, axis=1, keepdims=True)
    lanes = lax.broadcasted_iota(jnp.int32, x.shape, 1)
    onehot = lanes == t
    xt = jnp.sum(jnp.where(onehot, x, 0.0), axis=1, keepdims=True)
    st = jnp.sum(jnp.where(onehot, s, 0.0), axis=1, keepdims=True)
    pred = jnp.min(jnp.where(x == m, lanes, V), axis=1, keepdims=True)
    s1_ref[...] = t1 - V * z
    s2_ref[...] = t2 - _A * z
    lpt_ref[...] = xt - z
    st_ref[...] = st
    pred_ref[...] = pred


def _row_stats(x, t2, s2d):
    f = jax.ShapeDtypeStruct((N, 1), jnp.float32)
    return pl.pallas_call(
        _row_stats_body,
        grid=(N // _RB,),
        in_specs=[
            pl.BlockSpec((_RB, V), lambda i: (i, 0)),
            pl.BlockSpec((_RB, 1), lambda i: (i, 0)),
            pl.BlockSpec((1, V), lambda i: (0, 0)),
        ],
        out_specs=[pl.BlockSpec((_RB, 1), lambda i: (i, 0))] * 5,
        out_shape=[f, f, f, f, jax.ShapeDtypeStruct((N, 1), jnp.int32)],
    )(x, t2, s2d)


_LN = 16
_NT = 16
_NW = 32
_SL = N // _NW
_SEGC = N // _NT
_RPAD = N + 32


def _sc_body(t_hbm, pred_hbm, out_hbm, t_v, pred_v, tk_v, pk_v, segid_v,
             mr_v, pm_v, vp_v, vn_v, ed_v, es_v, estab_v, out_v, es_sh):
    cid = lax.axis_index("c")
    sid = lax.axis_index("s")
    k = sid
    iota = lax.iota(jnp.int32, _LN)

    pltpu.sync_copy(t_hbm, t_v.at[pl.ds(0, N)])
    pltpu.sync_copy(pred_hbm, pred_v.at[pl.ds(0, N)])

    def compact_body(c, carry):
        nrun, mrun = carry
        t16 = t_v[pl.ds(c * _LN, _LN)]
        p16 = pred_v[pl.ds(c * _LN, _LN)]
        mask = t16 != 0
        mask3 = t16 == 3
        mi = mask.astype(jnp.int32)
        m3i = mask3.astype(jnp.int32)
        cnt = jnp.max(plsc.all_reduce_population_count(mask))
        cnt3 = jnp.max(plsc.all_reduce_population_count(mask3))
        plsc.store_compressed(tk_v.at[pl.ds(nrun, _LN)], t16, mask=mask)
        plsc.store_compressed(pk_v.at[pl.ds(nrun, _LN)], p16, mask=mask)
        ranks = nrun + plsc.cumsum(mi) - 1
        plsc.store_compressed(mr_v.at[pl.ds(mrun, _LN)], ranks, mask=mask3)
        segid_v[pl.ds(c * _LN, _LN)] = mrun + plsc.cumsum(m3i) - m3i
        return nrun + cnt, mrun + cnt3

    _, nseg_total = lax.fori_loop(0, N // _LN, compact_body,
                                  (jnp.int32(0), jnp.int32(0)))

    def _sload(ref, idx):
        return plsc.load_gather(ref, [jnp.full((_LN,), idx, jnp.int32)])[0]

    wts = jnp.int32(1) << iota
    wts_hi = wts << 16
    one = jnp.int32(1)

    def seg_body(si, _):
        sgi = k + si * _NT
        prev_idx = jnp.maximum(sgi - 1, 0)
        bounds = plsc.load_gather(
            mr_v, [jnp.where(iota < 1, sgi, prev_idx)])
        end = bounds[0]
        start = jnp.where(sgi == 0, 0, bounds[1] + 1)
        seg_l = end - start
        mch = (seg_l + 511) >> 9
        mw = mch * _LN

        def zpm(c2, _c):
            pm_v[pl.ds(c2 * _LN, _LN)] = jnp.zeros((_LN,), jnp.int32)
            return 0

        lax.fori_loop(0, V * mch, zpm, 0)

        def bpm(i, _c):
            av = plsc.load_gather(
                tk_v, [jnp.full((_LN,), start + i, jnp.int32)])
            woff = ((i >> 5) & 15) + ((i >> 9) << 4)
            idxv = av * mw + woff
            w = plsc.load_gather(pm_v, [idxv])
            plsc.store_scatter(pm_v, [idxv], w | (one << (i & 31)))
            return 0

        lax.fori_loop(0, seg_l, bpm, 0)

        def ivp(cc, _c):
            bits = jnp.clip(seg_l - (cc * 512 + iota * 32), 0, 32)
            sh = jnp.minimum(bits, 31)
            mask = jnp.where(bits >= 32, jnp.int32(-1),
                             jnp.where(bits <= 0, jnp.int32(0),
                                       (one << sh) - 1))
            vp_v[pl.ds(cc * _LN, _LN)] = mask
            vn_v[pl.ds(cc * _LN, _LN)] = jnp.zeros((_LN,), jnp.int32)
            return 0

        lax.fori_loop(0, mch, ivp, 0)

        mm1 = seg_l - 1
        cm = mm1 >> 9
        selv = jnp.where(iota == ((mm1 >> 5) & 15),
                         one << (mm1 & 31), jnp.int32(0))

        def char_body(j, score):
            bj = _sload(pk_v, start + j)
            eqbase = bj * mw

            def chunk_body(cc, carry):
                score_c, addc, hpc, hnc = carry
                eq = pm_v[pl.ds(eqbase + cc * _LN, _LN)]
                vp = vp_v[pl.ds(cc * _LN, _LN)]
                vn = vn_v[pl.ds(cc * _LN, _LN)]
                a1 = eq & vp
                s = a1 + vp
                gv = (a1 & vp) | ((a1 | vp) & ~s)
                packed = (jnp.where(gv < 0, wts, 0)
                          | jnp.where(s == -1, wts_hi, 0))
                r = plsc.cumsum(packed)[15]
                g16 = r & 0xFFFF
                a16 = g16 | (lax.shift_right_logical(r, 16) & 0xFFFF)
                s16 = a16 + g16 + addc
                c16 = s16 ^ a16 ^ g16
                addc2 = lax.shift_right_logical(s16, 16) & 1
                s = s + (lax.shift_right_logical(
                    jnp.full((_LN,), c16, jnp.int32), iota) & 1)
                d0 = (s ^ vp) | eq | vn
                hp = vn | ~(d0 | vp)
                hn = vp & d0
                hpb = plsc.all_reduce_population_count((hp & selv) != 0)[0]
                hnb = plsc.all_reduce_population_count((hn & selv) != 0)[0]
                at_cm = jnp.where(cc == cm, one, jnp.int32(0))
                score_c2 = score_c + at_cm * (hpb - (1 - hpb) * hnb)
                packed2 = (jnp.where(hp < 0, wts, 0)
                           | jnp.where(hn < 0, wts_hi, 0))
                r2 = plsc.cumsum(packed2)[15]
                hpmsb = r2 & 0xFFFF
                hnmsb = lax.shift_right_logical(r2, 16) & 0xFFFF
                hp_in = ((hpmsb << 1) | hpc) & 0xFFFF
                hn_in = ((hnmsb << 1) | hnc) & 0xFFFF
                hpc2 = lax.shift_right_logical(hpmsb, 15) & 1
                hnc2 = lax.shift_right_logical(hnmsb, 15) & 1
                hps = (hp << 1) | (lax.shift_right_logical(
                    jnp.full((_LN,), hp_in, jnp.int32), iota) & 1)
                hns = (hn << 1) | (lax.shift_right_logical(
                    jnp.full((_LN,), hn_in, jnp.int32), iota) & 1)
                vp_v[pl.ds(cc * _LN, _LN)] = hns | ~(d0 | hps)
                vn_v[pl.ds(cc * _LN, _LN)] = hps & d0
                return score_c2, addc2, hpc2, hnc2

            sc2, _a, _b, _d = lax.fori_loop(
                0, mch, chunk_body,
                (score, jnp.int32(0), jnp.int32(1), jnp.int32(0)))
            return sc2

        ed = lax.fori_loop(0, seg_l, char_body, seg_l)
        plsc.store_scatter(ed_v, [jnp.full((_LN,), si, jnp.int32)],
                           jnp.full((_LN,), ed, jnp.int32))
        return 0

    nseg_mine = (jnp.maximum(nseg_total - k, 0) + _NT - 1) // _NT
    lax.fori_loop(0, nseg_mine, seg_body, 0)

    def es_body(c, _c):
        ed16 = ed_v[pl.ds(c * _LN, _LN)]
        edf = jnp.maximum(ed16.astype(jnp.float32), 1.0)
        es16 = jnp.where(ed16 != 0, 1.0 - jnp.exp(_LN09 / edf),
                         jnp.float32(1e-12))
        es_v[pl.ds(c * _LN, _LN)] = es16
        return 0

    lax.fori_loop(0, _SEGC // _LN, es_body, 0)
    pltpu.sync_copy(es_v, es_sh.at[k])
    plsc.subcore_barrier()

    pltpu.sync_copy(es_sh, estab_v)
    base = (cid * _NT + sid) * _SL

    def out_body(c, _c):
        off = base + c * _LN
        t16 = t_v[pl.ds(off, _LN)]
        sg16 = segid_v[pl.ds(off, _LN)]
        okseg = sg16 < nseg_total
        sgc = jnp.where(okseg, sg16, 0)
        es16 = plsc.load_gather(estab_v, [sgc & (_NT - 1), sgc >> 4])
        act = okseg & (t16 != 0) & (t16 != 3)
        out_v[pl.ds(c * _LN, _LN)] = jnp.where(act, es16, 0.0)
        return 0

    lax.fori_loop(0, _SL // _LN, out_body, 0)
    pltpu.sync_copy(out_v, out_hbm.at[pl.ds(base, _SL)])


@functools.cache
def _sc_es_call():
  return functools.partial(
    pl.kernel,
    out_type=jax.ShapeDtypeStruct((N,), jnp.float32),
    compiler_params=pltpu.CompilerParams(needs_layout_passes=False,
                                         use_tc_tiling_on_sc=False),
    mesh=plsc.VectorSubcoreMesh(core_axis_name="c", subcore_axis_name="s",
                                num_cores=2, num_subcores=16),
    scratch_types=[
        pltpu.VMEM((N + _LN,), jnp.int32),
        pltpu.VMEM((N + _LN,), jnp.int32),
        pltpu.VMEM((N + _LN,), jnp.int32),
        pltpu.VMEM((N + _LN,), jnp.int32),
        pltpu.VMEM((N,), jnp.int32),
        pltpu.VMEM((N,), jnp.int32),
        pltpu.VMEM((V * 24 * _LN,), jnp.int32),
        pltpu.VMEM((24 * _LN,), jnp.int32),
        pltpu.VMEM((24 * _LN,), jnp.int32),
        pltpu.VMEM((_SEGC,), jnp.int32),
        pltpu.VMEM((_SEGC,), jnp.float32),
        pltpu.VMEM((_NT, _SEGC), jnp.float32),
        pltpu.VMEM((_SL,), jnp.float32),
        pltpu.VMEM_SHARED((_NT, _SEGC), jnp.float32),
    ],
  )(_sc_body)


def _combine_body(t_ref, s1_ref, s2_ref, lpt_ref, st_ref, es_ref, out_ref):
    t = t_ref[...]
    s1 = s1_ref[...]
    s2 = s2_ref[...]
    lpt = lpt_ref[...]
    st = st_ref[...]
    es = es_ref[...]
    valid = t != 0
    active = es > 0.0
    head = (t >= 4) & (t < 24)
    ess = jnp.where(active, es, 1.0)
    vt = ess * st / (V - 1)
    ch = 1.0 - ess * _A / (V - 1)
    sum_vlogv = (ess / (V - 1)) * (_A * (jnp.log(ess) - _LOG_V1) + _B)
    l_head = (sum_vlogv - vt * jnp.log(vt)) \
        - ((ess / (V - 1)) * s2 - vt * lpt) \
        + ch * jnp.log(ch) - ch * lpt
    kk = ess * jnp.float32(1e-12) / (V - 1)
    cn = 1.0 - V * kk
    l_tail = (V - 1) * kk * jnp.log(kk) - kk * (s1 - lpt) \
        + cn * jnp.log(cn) - cn * lpt
    l_in = -lpt
    l_row = jnp.where(active, jnp.where(head, l_head, l_tail), l_in)
    l_row = jnp.where(valid, l_row, 0.0)
    n = jnp.sum(valid.astype(jnp.float32), axis=(0, 1), keepdims=True)
    out_ref[...] = jnp.sum(l_row, axis=(0, 1), keepdims=True) / n


def _combine(t2, s1, s2, lpt, st, es2):
    return pl.pallas_call(
        _combine_body,
        out_shape=jax.ShapeDtypeStruct((1, 1), jnp.float32),
    )(t2, s1, s2, lpt, st, es2)


def kernel(input, target):
    x = jnp.reshape(input, (N, V))
    t1 = jnp.reshape(target, (N,)).astype(jnp.int32)
    t2 = jnp.reshape(t1, (N, 1))
    s2d = jnp.asarray(_CLS_SMOOTH.reshape(1, V), dtype=jnp.float32)
    s1, s2, lpt, st, pred = _row_stats(x, t2, s2d)
    es = _sc_es_call()(t1, jnp.reshape(pred, (N,)))
    sq = lambda a: jnp.reshape(a, (96, 128))
    out = _combine(sq(t2), sq(s1), sq(s2), sq(lpt), sq(st), sq(es))
    return out[0, 0]


# speedup vs baseline: 506.4780x; 1.4728x over previous
import functools

import numpy as np
import jax
import jax.numpy as jnp
from jax import lax
from jax.experimental import pallas as pl
from jax.experimental.pallas import tpu as pltpu
from jax.experimental.pallas import tpu_sc as plsc

V = 100
N = 12288
ALPHA = 0.1
SMOOTH_TAIL = 0.01

_cls_num = np.array([100000 - 1000 * i for i in range(100)], dtype=np.float64)
_CLS_SMOOTH = (SMOOTH_TAIL + (ALPHA - SMOOTH_TAIL)
               * (_cls_num - _cls_num.min()) / (_cls_num.max() - _cls_num.min()))
_A = float(_CLS_SMOOTH.sum())
_B = float((_CLS_SMOOTH * np.log(_CLS_SMOOTH)).sum())
_LOG_V1 = float(np.log(V - 1))
_LN09 = float(np.log(1.0 - ALPHA))


_RB = 1024


def _row_stats_body(x_ref, t_ref, s_ref, s1_ref, s2_ref, lpt_ref, st_ref,
                    pred_ref):
    x = x_ref[...]
    t = t_ref[...]
    s = s_ref[...]
    m = jnp.max(x, axis=1, keepdims=True)
    z = m + jnp.log(jnp.sum(jnp.exp(x - m), axis=1, keepdims=True))
    t1 = jnp.sum(x, axis=1, keepdims=True)
    t2 = jnp.sum(x * s, axis=1, keepdims=True)
    lanes = lax.broadcasted_iota(jnp.int32, x.shape, 1)
    onehot = lanes == t
    xt = jnp.sum(jnp.where(onehot, x, 0.0), axis=1, keepdims=True)
    st = jnp.sum(jnp.where(onehot, s, 0.0), axis=1, keepdims=True)
    pred = jnp.min(jnp.where(x == m, lanes, V), axis=1, keepdims=True)
    s1_ref[...] = t1 - V * z
    s2_ref[...] = t2 - _A * z
    lpt_ref[...] = xt - z
    st_ref[...] = st
    pred_ref[...] = pred


def _row_stats(x, t2, s2d):
    f = jax.ShapeDtypeStruct((N, 1), jnp.float32)
    return pl.pallas_call(
        _row_stats_body,
        grid=(N // _RB,),
        in_specs=[
            pl.BlockSpec((_RB, V), lambda i: (i, 0)),
            pl.BlockSpec((_RB, 1), lambda i: (i, 0)),
            pl.BlockSpec((1, V), lambda i: (0, 0)),
        ],
        out_specs=[pl.BlockSpec((_RB, 1), lambda i: (i, 0))] * 5,
        out_shape=[f, f, f, f, jax.ShapeDtypeStruct((N, 1), jnp.int32)],
    )(x, t2, s2d)


_LN = 16
_NT = 16
_NW = 32
_SL = N // _NW
_SEGC = N // _NT
_RPAD = N + 32


def _sc_body(t_hbm, pred_hbm, out_hbm, t_v, pred_v, tk_v, pk_v, segid_v,
             mr_v, pm_v, vp_v, vn_v, ed_v, es_v, estab_v, out_v, es_sh):
    cid = lax.axis_index("c")
    sid = lax.axis_index("s")
    k = sid
    iota = lax.iota(jnp.int32, _LN)

    pltpu.sync_copy(t_hbm, t_v.at[pl.ds(0, N)])
    pltpu.sync_copy(pred_hbm, pred_v.at[pl.ds(0, N)])

    def compact_body(c, carry):
        nrun, mrun = carry
        t16 = t_v[pl.ds(c * _LN, _LN)]
        p16 = pred_v[pl.ds(c * _LN, _LN)]
        mask = t16 != 0
        mask3 = t16 == 3
        mi = mask.astype(jnp.int32)
        m3i = mask3.astype(jnp.int32)
        cnt = plsc.all_reduce_population_count(mask)[0]
        cnt3 = plsc.all_reduce_population_count(mask3)[0]
        plsc.store_compressed(tk_v.at[pl.ds(nrun, _LN)], t16, mask=mask)
        plsc.store_compressed(pk_v.at[pl.ds(nrun, _LN)], p16, mask=mask)
        ranks = nrun + plsc.cumsum(mi) - 1
        plsc.store_compressed(mr_v.at[pl.ds(mrun, _LN)], ranks, mask=mask3)
        segid_v[pl.ds(c * _LN, _LN)] = mrun + plsc.cumsum(m3i) - m3i
        return nrun + cnt, mrun + cnt3

    _, nseg_total = lax.fori_loop(0, N // _LN, compact_body,
                                  (jnp.int32(0), jnp.int32(0)))

    def _sload(ref, idx):
        return plsc.load_gather(ref, [jnp.full((_LN,), idx, jnp.int32)])[0]

    wts = jnp.int32(1) << iota
    wts_hi = wts << 16
    one = jnp.int32(1)

    def seg_body(si, _):
        sgi = k + si * _NT
        prev_idx = jnp.maximum(sgi - 1, 0)
        bounds = plsc.load_gather(
            mr_v, [jnp.where(iota < 1, sgi, prev_idx)])
        end = bounds[0]
        start = jnp.where(sgi == 0, 0, bounds[1] + 1)
        seg_l = end - start
        mch = (seg_l + 511) >> 9
        mw = mch * _LN

        def zpm(c2, _c):
            pm_v[pl.ds(c2 * _LN, _LN)] = jnp.zeros((_LN,), jnp.int32)
            return 0

        lax.fori_loop(0, V * mch, zpm, 0)

        def bpm(i, _c):
            av = plsc.load_gather(
                tk_v, [jnp.full((_LN,), start + i, jnp.int32)])
            woff = ((i >> 5) & 15) + ((i >> 9) << 4)
            idxv = av * mw + woff
            w = plsc.load_gather(pm_v, [idxv])
            plsc.store_scatter(pm_v, [idxv], w | (one << (i & 31)))
            return 0

        lax.fori_loop(0, seg_l, bpm, 0)

        def ivp(cc, _c):
            bits = jnp.clip(seg_l - (cc * 512 + iota * 32), 0, 32)
            sh = jnp.minimum(bits, 31)
            mask = jnp.where(bits >= 32, jnp.int32(-1),
                             jnp.where(bits <= 0, jnp.int32(0),
                                       (one << sh) - 1))
            vp_v[pl.ds(cc * _LN, _LN)] = mask
            vn_v[pl.ds(cc * _LN, _LN)] = jnp.zeros((_LN,), jnp.int32)
            return 0

        lax.fori_loop(0, mch, ivp, 0)

        mm1 = seg_l - 1
        cm = mm1 >> 9
        selv = jnp.where(iota == ((mm1 >> 5) & 15),
                         one << (mm1 & 31), jnp.int32(0))

        shm1 = jnp.maximum(iota - 1, 0)
        lane0 = iota == 0

        def fast_path(_):
            vp0 = vp_v[pl.ds(0, _LN)]
            zero16 = jnp.zeros((_LN,), jnp.int32)

            def cbody(j, carry):
                vp, vn, sv = carry
                bjv = plsc.load_gather(
                    pk_v, [jnp.full((_LN,), start + j, jnp.int32)])
                eq = plsc.load_gather(pm_v, [bjv * _LN + iota])
                a1 = eq & vp
                s = a1 + vp
                gv = (a1 & vp) | ((a1 | vp) & ~s)
                packed = (jnp.where(gv < 0, wts, 0)
                          | jnp.where(s == -1, wts_hi, 0))
                r_ex = plsc.cumsum(packed) - packed
                g16 = r_ex & 0xFFFF
                a16 = g16 | lax.shift_right_logical(r_ex, 16)
                cbits = (a16 + g16) ^ a16 ^ g16
                s = s + (lax.shift_right_logical(cbits, iota) & 1)
                d0 = (s ^ vp) | eq | vn
                hp = vn | ~(d0 | vp)
                hn = vp & d0
                sv = (sv + jnp.where((hp & selv) != 0, one, 0)
                      - jnp.where((hn & selv) != 0, one, 0))
                packed2 = (jnp.where(hp < 0, wts, 0)
                           | jnp.where(hn < 0, wts_hi, 0))
                r2ex = plsc.cumsum(packed2) - packed2
                hp_in = jnp.where(
                    lane0, one,
                    lax.shift_right_logical(r2ex & 0xFFFF, shm1) & 1)
                hn_in = lax.shift_right_logical(
                    lax.shift_right_logical(r2ex, 16), shm1) & 1
                hps = (hp << 1) | hp_in
                hns = (hn << 1) | hn_in
                return hns | ~(d0 | hps), hps & d0, sv

            _vp, _vn, sv = lax.fori_loop(
                0, seg_l, cbody, (vp0, zero16, zero16))
            return seg_l + plsc.cumsum(sv)[15]

        def char_body(j, score):
            bj = _sload(pk_v, start + j)
            eqbase = bj * mw

            def chunk_body(cc, carry):
                score_c, addc, hpc, hnc = carry
                eq = pm_v[pl.ds(eqbase + cc * _LN, _LN)]
                vp = vp_v[pl.ds(cc * _LN, _LN)]
                vn = vn_v[pl.ds(cc * _LN, _LN)]
                a1 = eq & vp
                s = a1 + vp
                gv = (a1 & vp) | ((a1 | vp) & ~s)
                packed = (jnp.where(gv < 0, wts, 0)
                          | jnp.where(s == -1, wts_hi, 0))
                r = plsc.cumsum(packed)[15]
                g16 = r & 0xFFFF
                a16 = g16 | (lax.shift_right_logical(r, 16) & 0xFFFF)
                s16 = a16 + g16 + addc
                c16 = s16 ^ a16 ^ g16
                addc2 = lax.shift_right_logical(s16, 16) & 1
                s = s + (lax.shift_right_logical(
                    jnp.full((_LN,), c16, jnp.int32), iota) & 1)
                d0 = (s ^ vp) | eq | vn
                hp = vn | ~(d0 | vp)
                hn = vp & d0
                hpb = plsc.all_reduce_population_count((hp & selv) != 0)[0]
                hnb = plsc.all_reduce_population_count((hn & selv) != 0)[0]
                at_cm = jnp.where(cc == cm, one, jnp.int32(0))
                score_c2 = score_c + at_cm * (hpb - (1 - hpb) * hnb)
                packed2 = (jnp.where(hp < 0, wts, 0)
                           | jnp.where(hn < 0, wts_hi, 0))
                r2 = plsc.cumsum(packed2)[15]
                hpmsb = r2 & 0xFFFF
                hnmsb = lax.shift_right_logical(r2, 16) & 0xFFFF
                hp_in = ((hpmsb << 1) | hpc) & 0xFFFF
                hn_in = ((hnmsb << 1) | hnc) & 0xFFFF
                hpc2 = lax.shift_right_logical(hpmsb, 15) & 1
                hnc2 = lax.shift_right_logical(hnmsb, 15) & 1
                hps = (hp << 1) | (lax.shift_right_logical(
                    jnp.full((_LN,), hp_in, jnp.int32), iota) & 1)
                hns = (hn << 1) | (lax.shift_right_logical(
                    jnp.full((_LN,), hn_in, jnp.int32), iota) & 1)
                vp_v[pl.ds(cc * _LN, _LN)] = hns | ~(d0 | hps)
                vn_v[pl.ds(cc * _LN, _LN)] = hps & d0
                return score_c2, addc2, hpc2, hnc2

            sc2, _a, _b, _d = lax.fori_loop(
                0, mch, chunk_body,
                (score, jnp.int32(0), jnp.int32(1), jnp.int32(0)))
            return sc2

        def general_path(_):
            return lax.fori_loop(0, seg_l, char_body, seg_l)

        ed = lax.cond(mch == 1, fast_path, general_path, 0)
        plsc.store_scatter(ed_v, [jnp.full((_LN,), si, jnp.int32)],
                           jnp.full((_LN,), ed, jnp.int32))
        return 0

    nseg_mine = (jnp.maximum(nseg_total - k, 0) + _NT - 1) // _NT
    lax.fori_loop(0, nseg_mine, seg_body, 0)

    def es_body(c, _c):
        ed16 = ed_v[pl.ds(c * _LN, _LN)]
        edf = jnp.maximum(ed16.astype(jnp.float32), 1.0)
        es16 = jnp.where(ed16 != 0, 1.0 - jnp.exp(_LN09 / edf),
                         jnp.float32(1e-12))
        es_v[pl.ds(c * _LN, _LN)] = es16
        return 0

    lax.fori_loop(0, _SEGC // _LN, es_body, 0)
    pltpu.sync_copy(es_v, es_sh.at[k])
    plsc.subcore_barrier()

    pltpu.sync_copy(es_sh, estab_v)
    base = (cid * _NT + sid) * _SL

    def out_body(c, _c):
        off = base + c * _LN
        t16 = t_v[pl.ds(off, _LN)]
        sg16 = segid_v[pl.ds(off, _LN)]
        okseg = sg16 < nseg_total
        sgc = jnp.where(okseg, sg16, 0)
        es16 = plsc.load_gather(estab_v, [sgc & (_NT - 1), sgc >> 4])
        act = okseg & (t16 != 0) & (t16 != 3)
        out_v[pl.ds(c * _LN, _LN)] = jnp.where(act, es16, 0.0)
        return 0

    lax.fori_loop(0, _SL // _LN, out_body, 0)
    pltpu.sync_copy(out_v, out_hbm.at[pl.ds(base, _SL)])


@functools.cache
def _sc_es_call():
  return functools.partial(
    pl.kernel,
    out_type=jax.ShapeDtypeStruct((N,), jnp.float32),
    compiler_params=pltpu.CompilerParams(needs_layout_passes=False,
                                         use_tc_tiling_on_sc=False),
    mesh=plsc.VectorSubcoreMesh(core_axis_name="c", subcore_axis_name="s",
                                num_cores=2, num_subcores=16),
    scratch_types=[
        pltpu.VMEM((N + _LN,), jnp.int32),
        pltpu.VMEM((N + _LN,), jnp.int32),
        pltpu.VMEM((N + _LN,), jnp.int32),
        pltpu.VMEM((N + _LN,), jnp.int32),
        pltpu.VMEM((N,), jnp.int32),
        pltpu.VMEM((N,), jnp.int32),
        pltpu.VMEM((V * 24 * _LN,), jnp.int32),
        pltpu.VMEM((24 * _LN,), jnp.int32),
        pltpu.VMEM((24 * _LN,), jnp.int32),
        pltpu.VMEM((_SEGC,), jnp.int32),
        pltpu.VMEM((_SEGC,), jnp.float32),
        pltpu.VMEM((_NT, _SEGC), jnp.float32),
        pltpu.VMEM((_SL,), jnp.float32),
        pltpu.VMEM_SHARED((_NT, _SEGC), jnp.float32),
    ],
  )(_sc_body)


def _combine_body(t_ref, s1_ref, s2_ref, lpt_ref, st_ref, es_ref, out_ref):
    t = t_ref[...]
    s1 = s1_ref[...]
    s2 = s2_ref[...]
    lpt = lpt_ref[...]
    st = st_ref[...]
    es = es_ref[...]
    valid = t != 0
    active = es > 0.0
    head = (t >= 4) & (t < 24)
    ess = jnp.where(active, es, 1.0)
    vt = ess * st / (V - 1)
    ch = 1.0 - ess * _A / (V - 1)
    sum_vlogv = (ess / (V - 1)) * (_A * (jnp.log(ess) - _LOG_V1) + _B)
    l_head = (sum_vlogv - vt * jnp.log(vt)) \
        - ((ess / (V - 1)) * s2 - vt * lpt) \
        + ch * jnp.log(ch) - ch * lpt
    kk = ess * jnp.float32(1e-12) / (V - 1)
    cn = 1.0 - V * kk
    l_tail = (V - 1) * kk * jnp.log(kk) - kk * (s1 - lpt) \
        + cn * jnp.log(cn) - cn * lpt
    l_in = -lpt
    l_row = jnp.where(active, jnp.where(head, l_head, l_tail), l_in)
    l_row = jnp.where(valid, l_row, 0.0)
    n = jnp.sum(valid.astype(jnp.float32), axis=(0, 1), keepdims=True)
    out_ref[...] = jnp.sum(l_row, axis=(0, 1), keepdims=True) / n


def _combine(t2, s1, s2, lpt, st, es2):
    return pl.pallas_call(
        _combine_body,
        out_shape=jax.ShapeDtypeStruct((1, 1), jnp.float32),
    )(t2, s1, s2, lpt, st, es2)


def kernel(input, target):
    x = jnp.reshape(input, (N, V))
    t1 = jnp.reshape(target, (N,)).astype(jnp.int32)
    t2 = jnp.reshape(t1, (N, 1))
    s2d = jnp.asarray(_CLS_SMOOTH.reshape(1, V), dtype=jnp.float32)
    s1, s2, lpt, st, pred = _row_stats(x, t2, s2d)
    es = _sc_es_call()(t1, jnp.reshape(pred, (N,)))
    sq = lambda a: jnp.reshape(a, (96, 128))
    out = _combine(sq(t2), sq(s1), sq(s2), sq(lpt), sq(st), sq(es))
    return out[0, 0]
